# Initial kernel scaffold; baseline (speedup 1.0000x reference)
import jax
import jax.numpy as jnp
from jax.experimental import pallas as pl


def kernel(inputs, tok_table, pos_table):
    raise NotImplementedError("write your pallas kernel here")


# speedup vs baseline: 1.3944x; 1.3944x over previous
import functools

import jax
import jax.numpy as jnp
from jax import lax
from jax.experimental import pallas as pl
from jax.experimental.pallas import tpu as pltpu
from jax.experimental.pallas import tpu_sc as plsc


def _make_sc_kernel(B, L, E, NC, NS):
    NW = NC * NS
    assert B % NW == 0
    RW = B // NW
    C = 4
    assert RW % C == 0
    NCHUNK = RW // C
    CL = C * L
    GS = 80
    assert CL % GS == 0 and GS % 8 == 0
    NG = CL // GS

    mesh = plsc.VectorSubcoreMesh(core_axis_name="c", subcore_axis_name="s")

    @functools.partial(
        pl.kernel,
        out_type=jax.ShapeDtypeStruct((B * L, E), jnp.float32),
        mesh=mesh,
        scratch_types=[
            pltpu.VMEM((L, E), jnp.float32),
            pltpu.VMEM((CL,), jnp.int32),
            pltpu.VMEM((CL, E), jnp.float32),
            pltpu.SemaphoreType.DMA,
        ],
        compiler_params=pltpu.CompilerParams(use_tc_tiling_on_sc=False),
    )
    def emb(tok_hbm, idx_hbm, pos_hbm, out_hbm, pos_v, idx_v, rows_v, gsem):
        wid = lax.axis_index("s") * NC + lax.axis_index("c")
        pltpu.sync_copy(pos_hbm, pos_v)
        base = wid * RW * L

        @pl.loop(0, NCHUNK)
        def _chunk(chunk):
            i0 = base + chunk * CL
            pltpu.sync_copy(idx_hbm.at[pl.ds(i0, CL)], idx_v)
            descs = [
                pltpu.async_copy(
                    tok_hbm.at[idx_v.at[pl.ds(k * GS, GS)]],
                    rows_v.at[pl.ds(k * GS, GS), :],
                    gsem,
                )
                for k in range(NG)
            ]
            for d in descs:
                d.wait()

            @pl.loop(0, L)
            def _row(j):
                p0 = pos_v[j, pl.ds(0, 16)]
                p1 = pos_v[j, pl.ds(16, 16)]
                for c in range(C):
                    r = c * L + j
                    rows_v[r, pl.ds(0, 16)] += p0
                    rows_v[r, pl.ds(16, 16)] += p1

            pltpu.sync_copy(rows_v, out_hbm.at[pl.ds(i0, CL), :])

    return emb


def kernel(inputs, tok_table, pos_table):
    B, L = inputs.shape
    E = tok_table.shape[1]
    info = plsc.get_sparse_core_info()
    emb = _make_sc_kernel(B, L, E, info.num_cores, info.num_subcores)
    idx_flat = inputs.reshape(-1).astype(jnp.int32)
    out = emb(tok_table, idx_flat, pos_table)
    return out.reshape(B, L, E)


# speedup vs baseline: 1.4865x; 1.0661x over previous
import functools

import jax
import jax.numpy as jnp
from jax import lax
from jax.experimental import pallas as pl
from jax.experimental.pallas import tpu as pltpu
from jax.experimental.pallas import tpu_sc as plsc


def _make_sc_kernel(B, L, E, NC, NS):
    NW = NC * NS
    assert B % NW == 0
    RW = B // NW
    C = 4
    assert RW % C == 0
    NCHUNK = RW // C
    CL = C * L
    WL = RW * L
    GS = 80
    assert CL % GS == 0 and GS % 8 == 0
    NG = CL // GS

    mesh = plsc.VectorSubcoreMesh(core_axis_name="c", subcore_axis_name="s")

    @functools.partial(
        pl.kernel,
        out_type=jax.ShapeDtypeStruct((B * L, E), jnp.float32),
        mesh=mesh,
        scratch_types=[
            pltpu.VMEM((L, E), jnp.float32),
            pltpu.VMEM((WL,), jnp.int32),
            pltpu.VMEM((2, CL, E), jnp.float32),
            pltpu.SemaphoreType.DMA((2,)),
            pltpu.SemaphoreType.DMA((2,)),
        ],
        compiler_params=pltpu.CompilerParams(use_tc_tiling_on_sc=False),
    )
    def emb(tok_hbm, idx_hbm, pos_hbm, out_hbm, pos_v, idx_v, rows_v, gsem, osem):
        wid = lax.axis_index("s") * NC + lax.axis_index("c")
        base = wid * WL
        pltpu.sync_copy(idx_hbm.at[pl.ds(base, WL)], idx_v)
        pltpu.sync_copy(pos_hbm, pos_v)

        def fire_gathers(g, s):
            return [
                pltpu.async_copy(
                    tok_hbm.at[idx_v.at[pl.ds(g * CL + k * GS, GS)]],
                    rows_v.at[s, pl.ds(k * GS, GS), :],
                    gsem.at[s],
                )
                for k in range(NG)
            ]

        gather_descs = {0: fire_gathers(0, 0)}
        out_descs = {}
        for g in range(NCHUNK):
            s = g % 2
            if g + 1 < NCHUNK:
                if g >= 1:
                    out_descs[g - 1].wait()
                gather_descs[g + 1] = fire_gathers(g + 1, 1 - s)
            for d in gather_descs.pop(g):
                d.wait()

            @pl.loop(0, L)
            def _row(j):
                p0 = pos_v[j, pl.ds(0, 16)]
                p1 = pos_v[j, pl.ds(16, 16)]
                for c in range(C):
                    r = c * L + j
                    rows_v[s, r, pl.ds(0, 16)] += p0
                    rows_v[s, r, pl.ds(16, 16)] += p1

            out_descs[g] = pltpu.async_copy(
                rows_v.at[s], out_hbm.at[pl.ds(base + g * CL, CL), :], osem.at[s]
            )
        out_descs[NCHUNK - 2].wait()
        out_descs[NCHUNK - 1].wait()

    return emb


def kernel(inputs, tok_table, pos_table):
    B, L = inputs.shape
    E = tok_table.shape[1]
    info = plsc.get_sparse_core_info()
    emb = _make_sc_kernel(B, L, E, info.num_cores, info.num_subcores)
    idx_flat = inputs.reshape(-1).astype(jnp.int32)
    out = emb(tok_table, idx_flat, pos_table)
    return out.reshape(B, L, E)
